# baseline (speedup 1.0000x reference)
import jax
import jax.numpy as jnp
from jax import lax
from jax.experimental import pallas as pl
from jax.experimental.pallas import tpu as pltpu
from jax.experimental.pallas import tpu_sc as plsc

_INFO = plsc.get_sparse_core_info()
_NC = _INFO.num_cores
_NS = _INFO.num_subcores
_L = _INFO.num_lanes
_NW = _NC * _NS

_BATCH = 16384
_D = 32
_BPW = _BATCH // _NW


def _collab_body(users_hbm, movies_hbm, eu_hbm, em_hbm, bu_hbm, bm_hbm,
                 out_hbm, idx_u, idx_m, rows_u, rows_m, bu_v, bm_v, out_v,
                 sem):
    wid = lax.axis_index("s") * _NC + lax.axis_index("c")
    base = wid * _BPW

    pltpu.sync_copy(users_hbm.at[pl.ds(base, _BPW)], idx_u)
    pltpu.sync_copy(movies_hbm.at[pl.ds(base, _BPW)], idx_m)

    cp1 = pltpu.async_copy(eu_hbm.at[idx_u], rows_u, sem)
    cp2 = pltpu.async_copy(em_hbm.at[idx_m], rows_m, sem)
    cp3 = pltpu.async_copy(bu_hbm.at[idx_u], bu_v, sem)
    cp4 = pltpu.async_copy(bm_hbm.at[idx_m], bm_v, sem)
    cp1.wait()
    cp2.wait()
    cp3.wait()
    cp4.wait()

    lanes = lax.iota(jnp.int32, _L)
    onehots = [lanes == k for k in range(_L)]
    shuf8 = (lanes + 8) % _L
    shuf4 = (lanes + 4) % _L
    shuf2 = (lanes + 2) % _L
    shuf1 = (lanes + 1) % _L

    dnums = lax.GatherDimensionNumbers(
        offset_dims=(), collapsed_slice_dims=(0,), start_index_map=(0,))

    def shuffle(t, idx):
        return lax.gather(t, idx[:, None], dnums, slice_sizes=(1,),
                          mode=lax.GatherScatterMode.PROMISE_IN_BOUNDS)

    def chunk(c, carry):
        b = c * _L
        dot = bu_v[pl.ds(b, _L)] + bm_v[pl.ds(b, _L)]
        for k in range(_L):
            r = b + k
            t = (rows_u[r, pl.ds(0, _L)] * rows_m[r, pl.ds(0, _L)] +
                 rows_u[r, pl.ds(_L, _L)] * rows_m[r, pl.ds(_L, _L)])
            t = t + shuffle(t, shuf8)
            t = t + shuffle(t, shuf4)
            t = t + shuffle(t, shuf2)
            t = t + shuffle(t, shuf1)
            dot = dot + jnp.where(onehots[k], t, 0.0)
        out_v[pl.ds(b, _L)] = 5.25 / (1.0 + jnp.exp(-dot))
        return carry

    lax.fori_loop(0, _BPW // _L, chunk, 0)
    pltpu.sync_copy(out_v, out_hbm.at[pl.ds(base, _BPW)])


def kernel(users, movies, embedding_user, embedding_movie, bias_user,
           bias_movie):
    mesh = plsc.VectorSubcoreMesh(core_axis_name="c", subcore_axis_name="s")
    run = pl.kernel(
        _collab_body,
        mesh=mesh,
        compiler_params=pltpu.CompilerParams(use_tc_tiling_on_sc=False),
        out_type=jax.ShapeDtypeStruct((_BATCH,), jnp.float32),
        scratch_types=[
            pltpu.VMEM((_BPW,), jnp.int32),
            pltpu.VMEM((_BPW,), jnp.int32),
            pltpu.VMEM((_BPW, _D), jnp.float32),
            pltpu.VMEM((_BPW, _D), jnp.float32),
            pltpu.VMEM((_BPW,), jnp.float32),
            pltpu.VMEM((_BPW,), jnp.float32),
            pltpu.VMEM((_BPW,), jnp.float32),
            pltpu.SemaphoreType.DMA,
        ],
    )
    one = (users[0] * 0 + 1).astype(jnp.float32)
    return run(users.astype(jnp.int32), movies.astype(jnp.int32),
               embedding_user * one, embedding_movie * one,
               bias_user, bias_movie)
